# baseline (speedup 1.0000x reference)
import functools

import jax
import jax.numpy as jnp
from jax import lax
from jax.experimental import pallas as pl
from jax.experimental.pallas import tpu as pltpu
from jax.experimental.pallas import tpu_sc as plsc

N_NODES = 10000
E_EDGES = 320000
IN_CH = 128
HID = 64
OUT = 64
K_HOPS = 10
ALPHA = 0.1

NC = 2
NS = 16
NW = NC * NS
LANES = 16

NPT = 640
N_PAD = NS * NPT
C_EDGE = 128
E_TOT = E_EDGES + N_NODES
N_CHUNK = -(-E_TOT // (NW * C_EDGE))
E_PAD = NW * N_CHUNK * C_EDGE
NBUF = 3
N_CHUNK_A = N_CHUNK + NBUF
NZC = NPT // C_EDGE


def _dense_tc_kernel(xc_ref, xn_ref, w1_ref, w2_ref, skip_ref, hid_ref):
    dn = (((1,), (1,)), ((), ()))
    w1 = w1_ref[...]
    w2 = w2_ref[...]
    hc = jnp.maximum(lax.dot_general(xc_ref[...], w1, dn,
                                     preferred_element_type=jnp.float32), 0.0)
    skip_ref[...] = lax.dot_general(hc, w2, dn,
                                    preferred_element_type=jnp.float32)
    hn = jnp.maximum(lax.dot_general(xn_ref[...], w1, dn,
                                     preferred_element_type=jnp.float32), 0.0)
    hid_ref[...] = lax.dot_general(hn, w2, dn,
                                   preferred_element_type=jnp.float32)


def _prep_tc_kernel(degp_ref, h0_ref, dinv_ref, g0_ref):
    deg = degp_ref[0] + degp_ref[1]
    d = deg[:, 0:1]
    dinv = jnp.where(d > 0.0, lax.rsqrt(jnp.maximum(d, 1e-12)), 0.0)
    dinv_b = jnp.broadcast_to(dinv, (N_PAD, OUT))
    dinv_ref[...] = dinv_b
    g0_ref[...] = dinv_b * h0_ref[...]


def _update_tc_kernel(p_ref, dinv_ref, h0_ref, g_ref):
    s = p_ref[0] + p_ref[1]
    dinv = dinv_ref[...]
    g_ref[...] = ((1.0 - ALPHA) * dinv * dinv * s
                  + ALPHA * dinv * h0_ref[...])


def _final_tc_kernel(p_ref, dinv_ref, h0_ref, skip_ref, out_ref):
    s = p_ref[0] + p_ref[1]
    out_ref[...] = ((1.0 - ALPHA) * dinv_ref[...] * s
                    + ALPHA * h0_ref[...] + skip_ref[...])


def _sc_deg_body(row_hbm, degp_hbm,
                 ridx0, ridx1, ridx2, ones_v, zsm, is0, is1, is2, deg_sh):
    cid = lax.axis_index("c")
    sid = lax.axis_index("s")
    nbase = sid * NPT
    ridxs = (ridx0, ridx1, ridx2)
    isems = (is0, is1, is2)

    ones16 = jnp.ones((LANES,), jnp.float32)
    zero16 = jnp.zeros((LANES,), jnp.float32)

    def fill_const(i, _):
        ones_v[i, :] = ones16
        zsm[i, :] = zero16
        return 0
    lax.fori_loop(0, C_EDGE, fill_const, 0)

    for z in range(NZC):
        pltpu.sync_copy(zsm, deg_sh.at[pl.ds(nbase + z * C_EDGE, C_EDGE)])
    plsc.subcore_barrier()

    for b in range(NBUF):
        pltpu.async_copy(row_hbm.at[cid, sid, b], ridxs[b], isems[b])

    def trio(jj, _):
        j0 = jj * NBUF
        for b in range(NBUF):
            pltpu.make_async_copy(
                row_hbm.at[cid, sid, 0], ridxs[b], isems[b]).wait()
            pltpu.sync_copy(ones_v, deg_sh.at[ridxs[b]], add=True)
            pltpu.async_copy(
                row_hbm.at[cid, sid, j0 + b + NBUF], ridxs[b], isems[b])
        return 0
    lax.fori_loop(0, N_CHUNK // NBUF, trio, 0)

    for b in range(NBUF):
        pltpu.make_async_copy(row_hbm.at[cid, sid, 0], ridxs[b],
                              isems[b]).wait()
    plsc.subcore_barrier()

    pltpu.sync_copy(deg_sh.at[pl.ds(nbase, NPT)],
                    degp_hbm.at[cid, pl.ds(nbase, NPT)])


def _sc_scatter_body(g_hbm, row_hbm, col_hbm, part_hbm,
                     cidx0, cidx1, cidx2, ridx0, ridx1, ridx2,
                     gb0, gb1, gb2, zsm,
                     is0, is1, is2, gs0, gs1, gs2, ss0, ss1, ss2,
                     rs0, rs1, rs2, acc_sh):
    cid = lax.axis_index("c")
    sid = lax.axis_index("s")
    nbase = sid * NPT
    cidxs = (cidx0, cidx1, cidx2)
    ridxs = (ridx0, ridx1, ridx2)
    gbufs = (gb0, gb1, gb2)
    isems = (is0, is1, is2)
    gsems = (gs0, gs1, gs2)
    ssems = (ss0, ss1, ss2)
    rsems = (rs0, rs1, rs2)

    zero16 = jnp.zeros((LANES,), jnp.float32)

    def fill_zero(i, _):
        for k in range(OUT // LANES):
            zsm[i, pl.ds(k * LANES, LANES)] = zero16
        return 0
    lax.fori_loop(0, C_EDGE, fill_zero, 0)

    for z in range(NZC):
        pltpu.sync_copy(zsm, acc_sh.at[pl.ds(nbase + z * C_EDGE, C_EDGE)])
    plsc.subcore_barrier()

    for b in range(NBUF):
        pltpu.async_copy(col_hbm.at[cid, sid, b], cidxs[b], isems[b])
        pltpu.async_copy(row_hbm.at[cid, sid, b], ridxs[b], rsems[b])
    for b in range(NBUF):
        pltpu.make_async_copy(col_hbm.at[cid, sid, 0], cidxs[b],
                              isems[b]).wait()
        pltpu.async_copy(g_hbm.at[cidxs[b]], gbufs[b], gsems[b])

    def trio(jj, _):
        j0 = jj * NBUF
        for b in range(NBUF):
            pltpu.make_async_copy(g_hbm.at[cidxs[b]], gbufs[b],
                                  gsems[b]).wait()
            pltpu.make_async_copy(row_hbm.at[cid, sid, 0], ridxs[b],
                                  rsems[b]).wait()
            pltpu.async_copy(gbufs[b], acc_sh.at[ridxs[b]], ssems[b],
                             add=True)
            pltpu.async_copy(col_hbm.at[cid, sid, j0 + b + NBUF], cidxs[b],
                             isems[b])
        for b in range(NBUF):
            pltpu.make_async_copy(gbufs[b], acc_sh.at[ridxs[b]],
                                  ssems[b]).wait()
            pltpu.async_copy(row_hbm.at[cid, sid, j0 + b + NBUF], ridxs[b],
                             rsems[b])
            pltpu.make_async_copy(col_hbm.at[cid, sid, 0], cidxs[b],
                                  isems[b]).wait()
            pltpu.async_copy(g_hbm.at[cidxs[b]], gbufs[b], gsems[b])
        return 0
    lax.fori_loop(0, N_CHUNK // NBUF, trio, 0)

    for b in range(NBUF):
        pltpu.make_async_copy(g_hbm.at[cidxs[b]], gbufs[b], gsems[b]).wait()
        pltpu.make_async_copy(row_hbm.at[cid, sid, 0], ridxs[b],
                              rsems[b]).wait()
    plsc.subcore_barrier()

    pltpu.sync_copy(acc_sh.at[pl.ds(nbase, NPT)],
                    part_hbm.at[cid, pl.ds(nbase, NPT)])


_MESH = plsc.VectorSubcoreMesh(core_axis_name="c", subcore_axis_name="s")
_SC_PARAMS = pltpu.CompilerParams(use_tc_tiling_on_sc=False)

_deg_call = pl.kernel(
    _sc_deg_body,
    out_type=jax.ShapeDtypeStruct((NC, N_PAD, LANES), jnp.float32),
    mesh=_MESH,
    compiler_params=_SC_PARAMS,
    scratch_types=[
        pltpu.VMEM((C_EDGE,), jnp.int32),
        pltpu.VMEM((C_EDGE,), jnp.int32),
        pltpu.VMEM((C_EDGE,), jnp.int32),
        pltpu.VMEM((C_EDGE, LANES), jnp.float32),
        pltpu.VMEM((C_EDGE, LANES), jnp.float32),
        pltpu.SemaphoreType.DMA,
        pltpu.SemaphoreType.DMA,
        pltpu.SemaphoreType.DMA,
        pltpu.VMEM_SHARED((N_PAD, LANES), jnp.float32),
    ],
)

_scatter_call = pl.kernel(
    _sc_scatter_body,
    out_type=jax.ShapeDtypeStruct((NC, N_PAD, OUT), jnp.float32),
    mesh=_MESH,
    compiler_params=_SC_PARAMS,
    scratch_types=(
        [pltpu.VMEM((C_EDGE,), jnp.int32)] * 6
        + [pltpu.VMEM((C_EDGE, OUT), jnp.float32)] * 3
        + [pltpu.VMEM((C_EDGE, OUT), jnp.float32)]
        + [pltpu.SemaphoreType.DMA] * 12
        + [pltpu.VMEM_SHARED((N_PAD, OUT), jnp.float32)]
    ),
)


def kernel(x_clean, x_noised, edge_index, W1, W2):
    skip, hidden = pl.pallas_call(
        _dense_tc_kernel,
        out_shape=[
            jax.ShapeDtypeStruct((N_NODES, OUT), jnp.float32),
            jax.ShapeDtypeStruct((N_NODES, OUT), jnp.float32),
        ],
    )(x_clean, x_noised, W1, W2)

    loop = jnp.arange(N_NODES, dtype=jnp.int32)
    pad = jnp.full((E_PAD - E_TOT,), N_NODES, dtype=jnp.int32)
    tail = jnp.full((NC, NS, NBUF, C_EDGE), N_NODES, dtype=jnp.int32)
    row = jnp.concatenate([edge_index[0], loop, pad]).reshape(
        NC, NS, N_CHUNK, C_EDGE)
    col = jnp.concatenate([edge_index[1], loop, pad]).reshape(
        NC, NS, N_CHUNK, C_EDGE)
    row = jnp.concatenate([row, tail], axis=2)
    col = jnp.concatenate([col, tail], axis=2)

    pad_rows = jnp.zeros((N_PAD - N_NODES, OUT), jnp.float32)
    h0 = jnp.concatenate([hidden, pad_rows])
    skp = jnp.concatenate([skip, pad_rows])

    degp = _deg_call(row)
    dinv_b, g = pl.pallas_call(
        _prep_tc_kernel,
        out_shape=[
            jax.ShapeDtypeStruct((N_PAD, OUT), jnp.float32),
            jax.ShapeDtypeStruct((N_PAD, OUT), jnp.float32),
        ],
    )(degp, h0)

    upd = pl.pallas_call(
        _update_tc_kernel,
        out_shape=jax.ShapeDtypeStruct((N_PAD, OUT), jnp.float32),
    )
    for _ in range(K_HOPS - 1):
        part = _scatter_call(g, row, col)
        g = upd(part, dinv_b, h0)
    part = _scatter_call(g, row, col)

    out_pad = pl.pallas_call(
        _final_tc_kernel,
        out_shape=jax.ShapeDtypeStruct((N_PAD, OUT), jnp.float32),
    )(part, dinv_b, h0, skp)
    return out_pad[:N_NODES]
